# Initial kernel scaffold; baseline (speedup 1.0000x reference)
import jax
import jax.numpy as jnp
from jax.experimental import pallas as pl


def kernel(pred, prob):
    raise NotImplementedError("write your pallas kernel here")


# speedup vs baseline: 1.1992x; 1.1992x over previous
import functools

import jax
import jax.numpy as jnp
from jax import lax
from jax.experimental import pallas as pl
from jax.experimental.pallas import tpu as pltpu
from jax.experimental.pallas import tpu_sc as plsc

_C = 19
_P = 512 * 512
_NW = 32
_PW = _P // _NW
_CHUNK = 2048
_NCHUNK = _PW // _CHUNK
_GROUPS = _CHUNK // 16
_RATIO = 0.2
_IGNORE = -1.0


def _sc_body(prob_hbm, cnt_out, s2_out, buf, acc_cnt, acc_s2, obuf_cnt, obuf_s2):
    wid = lax.axis_index("s") * 2 + lax.axis_index("c")
    zero16 = jnp.zeros((16,), jnp.float32)
    for j in range(32):
        acc_cnt[pl.ds(j * 16, 16)] = zero16
        acc_s2[pl.ds(j * 16, 16)] = zero16
    lane = lax.iota(jnp.int32, 16)
    ones16 = jnp.ones((16,), jnp.float32)
    base_px = wid * _PW

    for ci in range(_NCHUNK):
        pltpu.sync_copy(prob_hbm.at[:, pl.ds(base_px + ci * _CHUNK, _CHUNK)], buf)

        def group_body(g, carry):
            off = g * 16
            v = buf[0, pl.ds(off, 16)]
            m = v
            a = jnp.zeros((16,), jnp.int32)
            s2 = v * v
            for c in range(1, _C):
                v = buf[c, pl.ds(off, 16)]
                gt = v > m
                m = jnp.where(gt, v, m)
                a = jnp.where(gt, c, a)
                s2 = s2 + v * v
            valid = m != _IGNORE
            addr = a * 16 + lane
            plsc.addupdate_scatter(acc_cnt, [addr], ones16, mask=valid)
            plsc.addupdate_scatter(acc_s2, [addr], s2, mask=valid)
            return carry

        lax.fori_loop(0, _GROUPS, group_body, 0)

    for half in range(2):
        kidx = (lane + half * 16) * 16
        csum = jnp.zeros((16,), jnp.float32)
        ssum = jnp.zeros((16,), jnp.float32)
        for j in range(16):
            csum = csum + plsc.load_gather(acc_cnt, [kidx + j])
            ssum = ssum + plsc.load_gather(acc_s2, [kidx + j])
        obuf_cnt[pl.ds(half * 16, 16)] = csum
        obuf_s2[pl.ds(half * 16, 16)] = ssum
    pltpu.sync_copy(obuf_cnt, cnt_out.at[wid])
    pltpu.sync_copy(obuf_s2, s2_out.at[wid])


def _fin_body(cnt_ref, s2_ref, out_ref):
    h = jnp.sum(cnt_ref[...], axis=0, keepdims=True)
    s = jnp.sum(s2_ref[...], axis=0, keepdims=True)
    col = lax.broadcasted_iota(jnp.int32, (1, 32), 1)
    validc = col < _C
    h = jnp.where(validc, h, 0.0)
    s = jnp.where(validc, s, 0.0)
    total = jnp.sum(h, keepdims=True)
    denom = jnp.maximum(
        jnp.power(h, _RATIO) * jnp.power(total, 1.0 - _RATIO), 1.0
    )
    out_ref[...] = -jnp.sum(s / denom, keepdims=True) / _C


def kernel(pred, prob):
    del pred
    prob2 = prob.reshape(_C, _P)
    mesh = plsc.VectorSubcoreMesh(core_axis_name="c", subcore_axis_name="s")
    sc = pl.kernel(
        _sc_body,
        out_type=[
            jax.ShapeDtypeStruct((_NW, 32), jnp.float32),
            jax.ShapeDtypeStruct((_NW, 32), jnp.float32),
        ],
        mesh=mesh,
        compiler_params=pltpu.CompilerParams(needs_layout_passes=False),
        scratch_types=[
            pltpu.VMEM((_C, _CHUNK), jnp.float32),
            pltpu.VMEM((512,), jnp.float32),
            pltpu.VMEM((512,), jnp.float32),
            pltpu.VMEM((32,), jnp.float32),
            pltpu.VMEM((32,), jnp.float32),
        ],
    )
    cnt, s2 = sc(prob2)
    loss = pl.pallas_call(
        _fin_body,
        out_shape=jax.ShapeDtypeStruct((1, 1), jnp.float32),
    )(cnt, s2)
    return loss.reshape(())


# speedup vs baseline: 2.0535x; 1.7124x over previous
import functools

import jax
import jax.numpy as jnp
from jax import lax
from jax.experimental import pallas as pl
from jax.experimental.pallas import tpu as pltpu
from jax.experimental.pallas import tpu_sc as plsc

_C = 19
_H = 512
_W = 512
_NW = 32
_RW = _H // _NW
_RCH = 4
_NCHUNK = _RW // _RCH
_GROUPS = (_RCH * _W) // 16
_RATIO = 0.2
_IGNORE = -1.0


def _sc_body(prob_hbm, cnt_out, s2_out, buf, acc_cnt, acc_s2, obuf_cnt, obuf_s2):
    wid = lax.axis_index("s") * 2 + lax.axis_index("c")
    zero16 = jnp.zeros((16,), jnp.float32)
    for j in range(32):
        acc_cnt[pl.ds(j * 16, 16)] = zero16
        acc_s2[pl.ds(j * 16, 16)] = zero16
    lane = lax.iota(jnp.int32, 16)
    ones16 = jnp.ones((16,), jnp.float32)
    base_row = wid * _RW

    for ci in range(_NCHUNK):
        pltpu.sync_copy(
            prob_hbm.at[0, :, pl.ds(base_row + ci * _RCH, _RCH), :], buf
        )

        for r in range(_RCH):
            def group_body(g, carry):
                off = g * 16
                v = buf[0, r, pl.ds(off, 16)]
                m = v
                a = jnp.zeros((16,), jnp.int32)
                s2 = v * v
                for c in range(1, _C):
                    v = buf[c, r, pl.ds(off, 16)]
                    gt = v > m
                    m = jnp.where(gt, v, m)
                    a = jnp.where(gt, c, a)
                    s2 = s2 + v * v
                valid = m != _IGNORE
                addr = a * 16 + lane
                plsc.addupdate_scatter(acc_cnt, [addr], ones16, mask=valid)
                plsc.addupdate_scatter(acc_s2, [addr], s2, mask=valid)
                return carry

            lax.fori_loop(0, _W // 16, group_body, 0)

    for half in range(2):
        kidx = (lane + half * 16) * 16
        csum = jnp.zeros((16,), jnp.float32)
        ssum = jnp.zeros((16,), jnp.float32)
        for j in range(16):
            csum = csum + plsc.load_gather(acc_cnt, [kidx + j])
            ssum = ssum + plsc.load_gather(acc_s2, [kidx + j])
        obuf_cnt[pl.ds(half * 16, 16)] = csum
        obuf_s2[pl.ds(half * 16, 16)] = ssum
    pltpu.sync_copy(obuf_cnt, cnt_out.at[wid])
    pltpu.sync_copy(obuf_s2, s2_out.at[wid])


def _fin_body(cnt_ref, s2_ref, out_ref):
    h = jnp.sum(cnt_ref[...], axis=0, keepdims=True)
    s = jnp.sum(s2_ref[...], axis=0, keepdims=True)
    col = lax.broadcasted_iota(jnp.int32, (1, 32), 1)
    validc = col < _C
    h = jnp.where(validc, h, 0.0)
    s = jnp.where(validc, s, 0.0)
    total = jnp.sum(h, keepdims=True)
    denom = jnp.maximum(
        jnp.power(h, _RATIO) * jnp.power(total, 1.0 - _RATIO), 1.0
    )
    out_ref[...] = -jnp.sum(s / denom, keepdims=True) / _C


def kernel(pred, prob):
    del pred
    mesh = plsc.VectorSubcoreMesh(core_axis_name="c", subcore_axis_name="s")
    sc = pl.kernel(
        _sc_body,
        out_type=[
            jax.ShapeDtypeStruct((_NW, 32), jnp.float32),
            jax.ShapeDtypeStruct((_NW, 32), jnp.float32),
        ],
        mesh=mesh,
        compiler_params=pltpu.CompilerParams(needs_layout_passes=False),
        scratch_types=[
            pltpu.VMEM((_C, _RCH, _W), jnp.float32),
            pltpu.VMEM((512,), jnp.float32),
            pltpu.VMEM((512,), jnp.float32),
            pltpu.VMEM((32,), jnp.float32),
            pltpu.VMEM((32,), jnp.float32),
        ],
    )
    cnt, s2 = sc(prob)
    loss = pl.pallas_call(
        _fin_body,
        out_shape=jax.ShapeDtypeStruct((1, 1), jnp.float32),
    )(cnt, s2)
    return loss.reshape(())


# speedup vs baseline: 2.2874x; 1.1139x over previous
import functools

import jax
import jax.numpy as jnp
from jax import lax
from jax.experimental import pallas as pl
from jax.experimental.pallas import tpu as pltpu
from jax.experimental.pallas import tpu_sc as plsc

_C = 19
_H = 512
_W = 512
_NW = 32
_RW = _H // _NW
_RCH = 4
_NCHUNK = _RW // _RCH
_GROUPS = (_RCH * _W) // 16
_RATIO = 0.2
_IGNORE = -1.0


def _sc_body(prob_hbm, cnt_out, s2_out, buf0, buf1, acc_cnt, acc_s2,
             obuf_cnt, obuf_s2, sem0, sem1):
    wid = lax.axis_index("s") * 2 + lax.axis_index("c")
    zero16 = jnp.zeros((16,), jnp.float32)
    for j in range(32):
        acc_cnt[pl.ds(j * 16, 16)] = zero16
        acc_s2[pl.ds(j * 16, 16)] = zero16
    lane = lax.iota(jnp.int32, 16)
    ones16 = jnp.ones((16,), jnp.float32)
    base_row = wid * _RW

    bufs = (buf0, buf1)
    sems = (sem0, sem1)

    def copy_of(ci):
        return pltpu.make_async_copy(
            prob_hbm.at[0, :, pl.ds(base_row + ci * _RCH, _RCH), :],
            bufs[ci % 2],
            sems[ci % 2],
        )

    def process16(buf, r, off):
        vals = [buf[c, r, pl.ds(off, 16)] for c in range(_C)]
        sq = [v * v for v in vals]
        while len(sq) > 1:
            nxt = [sq[i] + sq[i + 1] for i in range(0, len(sq) - 1, 2)]
            if len(sq) % 2:
                nxt.append(sq[-1])
            sq = nxt
        s2 = sq[0]
        ent = [(vals[c], c) for c in range(_C)]
        while len(ent) > 1:
            nxt = []
            for i in range(0, len(ent) - 1, 2):
                ml, al = ent[i]
                mr, ar = ent[i + 1]
                gt = mr > ml
                nxt.append((jnp.where(gt, mr, ml), jnp.where(gt, ar, al)))
            if len(ent) % 2:
                nxt.append(ent[-1])
            ent = nxt
        m, a = ent[0]
        valid = m != _IGNORE
        addr = a * 16 + lane
        plsc.addupdate_scatter(acc_cnt, [addr], ones16, mask=valid)
        plsc.addupdate_scatter(acc_s2, [addr], s2, mask=valid)

    copy_of(0).start()
    for ci in range(_NCHUNK):
        if ci + 1 < _NCHUNK:
            copy_of(ci + 1).start()
        copy_of(ci).wait()
        buf = bufs[ci % 2]
        for r in range(_RCH):
            def group_body(g, carry):
                off = g * 32
                process16(buf, r, off)
                process16(buf, r, off + 16)
                return carry

            lax.fori_loop(0, _W // 32, group_body, 0)

    for half in range(2):
        kidx = (lane + half * 16) * 16
        csum = jnp.zeros((16,), jnp.float32)
        ssum = jnp.zeros((16,), jnp.float32)
        for j in range(16):
            csum = csum + plsc.load_gather(acc_cnt, [kidx + j])
            ssum = ssum + plsc.load_gather(acc_s2, [kidx + j])
        obuf_cnt[pl.ds(half * 16, 16)] = csum
        obuf_s2[pl.ds(half * 16, 16)] = ssum
    pltpu.sync_copy(obuf_cnt, cnt_out.at[wid])
    pltpu.sync_copy(obuf_s2, s2_out.at[wid])


def _fin_body(cnt_ref, s2_ref, out_ref):
    h = jnp.sum(cnt_ref[...], axis=0, keepdims=True)
    s = jnp.sum(s2_ref[...], axis=0, keepdims=True)
    col = lax.broadcasted_iota(jnp.int32, (1, 32), 1)
    validc = col < _C
    h = jnp.where(validc, h, 0.0)
    s = jnp.where(validc, s, 0.0)
    total = jnp.sum(h, keepdims=True)
    denom = jnp.maximum(
        jnp.power(h, _RATIO) * jnp.power(total, 1.0 - _RATIO), 1.0
    )
    out_ref[...] = -jnp.sum(s / denom, keepdims=True) / _C


def kernel(pred, prob):
    del pred
    mesh = plsc.VectorSubcoreMesh(core_axis_name="c", subcore_axis_name="s")
    sc = pl.kernel(
        _sc_body,
        out_type=[
            jax.ShapeDtypeStruct((_NW, 32), jnp.float32),
            jax.ShapeDtypeStruct((_NW, 32), jnp.float32),
        ],
        mesh=mesh,
        compiler_params=pltpu.CompilerParams(needs_layout_passes=False),
        scratch_types=[
            pltpu.VMEM((_C, _RCH, _W), jnp.float32),
            pltpu.VMEM((_C, _RCH, _W), jnp.float32),
            pltpu.VMEM((512,), jnp.float32),
            pltpu.VMEM((512,), jnp.float32),
            pltpu.VMEM((32,), jnp.float32),
            pltpu.VMEM((32,), jnp.float32),
            pltpu.SemaphoreType.DMA,
            pltpu.SemaphoreType.DMA,
        ],
    )
    cnt, s2 = sc(prob)
    loss = pl.pallas_call(
        _fin_body,
        out_shape=jax.ShapeDtypeStruct((1, 1), jnp.float32),
    )(cnt, s2)
    return loss.reshape(())


# speedup vs baseline: 2.9751x; 1.3006x over previous
import functools

import jax
import jax.numpy as jnp
from jax import lax
from jax.experimental import pallas as pl
from jax.experimental.pallas import tpu as pltpu
from jax.experimental.pallas import tpu_sc as plsc

_C = 19
_H = 512
_W = 512
_NW = 32
_RSC = 256
_RW = _RSC // _NW
_RCH = 2
_NCHUNK = _RW // _RCH
_HB = 64
_RATIO = 0.2
_IGNORE = -1.0


def _sc_body(prob_hbm, cnt_out, s2_out, buf0, buf1, acc_cnt, acc_s2,
             obuf_cnt, obuf_s2, sem0, sem1):
    wid = lax.axis_index("s") * 2 + lax.axis_index("c")
    zero16 = jnp.zeros((16,), jnp.float32)
    for j in range(32):
        acc_cnt[pl.ds(j * 16, 16)] = zero16
        acc_s2[pl.ds(j * 16, 16)] = zero16
    lane = lax.iota(jnp.int32, 16)
    ones16 = jnp.ones((16,), jnp.float32)
    base_row = wid * _RW

    bufs = (buf0, buf1)
    sems = (sem0, sem1)

    def copy_of(ci):
        return pltpu.make_async_copy(
            prob_hbm.at[0, :, pl.ds(base_row + ci * _RCH, _RCH), :],
            bufs[ci % 2],
            sems[ci % 2],
        )

    def process16(buf, r, off):
        vals = [buf[c, r, pl.ds(off, 16)] for c in range(_C)]
        sq = [v * v for v in vals]
        while len(sq) > 1:
            nxt = [sq[i] + sq[i + 1] for i in range(0, len(sq) - 1, 2)]
            if len(sq) % 2:
                nxt.append(sq[-1])
            sq = nxt
        s2 = sq[0]
        ent = [(vals[c], c) for c in range(_C)]
        while len(ent) > 1:
            nxt = []
            for i in range(0, len(ent) - 1, 2):
                ml, al = ent[i]
                mr, ar = ent[i + 1]
                gt = mr > ml
                nxt.append((jnp.where(gt, mr, ml), jnp.where(gt, ar, al)))
            if len(ent) % 2:
                nxt.append(ent[-1])
            ent = nxt
        m, a = ent[0]
        valid = m != _IGNORE
        addr = a * 16 + lane
        plsc.addupdate_scatter(acc_cnt, [addr], ones16, mask=valid)
        plsc.addupdate_scatter(acc_s2, [addr], s2, mask=valid)

    copy_of(0).start()
    for ci in range(_NCHUNK):
        if ci + 1 < _NCHUNK:
            copy_of(ci + 1).start()
        copy_of(ci).wait()
        buf = bufs[ci % 2]
        for r in range(_RCH):
            def group_body(g, carry):
                off = g * 32
                process16(buf, r, off)
                process16(buf, r, off + 16)
                return carry

            lax.fori_loop(0, _W // 32, group_body, 0)

    for half in range(2):
        kidx = (lane + half * 16) * 16
        csum = jnp.zeros((16,), jnp.float32)
        ssum = jnp.zeros((16,), jnp.float32)
        for j in range(16):
            csum = csum + plsc.load_gather(acc_cnt, [kidx + j])
            ssum = ssum + plsc.load_gather(acc_s2, [kidx + j])
        obuf_cnt[pl.ds(half * 16, 16)] = csum
        obuf_s2[pl.ds(half * 16, 16)] = ssum
    pltpu.sync_copy(obuf_cnt, cnt_out.at[wid])
    pltpu.sync_copy(obuf_s2, s2_out.at[wid])


def _tc_hist_body(x_ref, out_ref):
    v = x_ref[0, 0]
    m = v
    a = jnp.zeros((_HB, _W), jnp.int32)
    s2 = v * v
    for c in range(1, _C):
        v = x_ref[0, c]
        gt = v > m
        m = jnp.where(gt, v, m)
        a = jnp.where(gt, c, a)
        s2 = s2 + v * v
    valid = m != _IGNORE
    col = lax.broadcasted_iota(jnp.int32, (1, 32), 1)
    acc_c = jnp.zeros((1, 32), jnp.float32)
    acc_s = jnp.zeros((1, 32), jnp.float32)
    for k in range(_C):
        mk = jnp.logical_and(a == k, valid)
        ck = jnp.sum(jnp.where(mk, 1.0, 0.0))
        sk = jnp.sum(jnp.where(mk, s2, 0.0))
        acc_c = acc_c + jnp.where(col == k, ck, 0.0)
        acc_s = acc_s + jnp.where(col == k, sk, 0.0)

    @pl.when(pl.program_id(0) == 0)
    def _():
        out_ref[...] = jnp.zeros_like(out_ref)

    out_ref[0:1, :] = out_ref[0:1, :] + acc_c
    out_ref[1:2, :] = out_ref[1:2, :] + acc_s


def _fin_body(cnt_ref, s2_ref, tc_ref, out_ref):
    h = jnp.sum(cnt_ref[...], axis=0, keepdims=True) + tc_ref[0:1, :]
    s = jnp.sum(s2_ref[...], axis=0, keepdims=True) + tc_ref[1:2, :]
    col = lax.broadcasted_iota(jnp.int32, (1, 32), 1)
    validc = col < _C
    h = jnp.where(validc, h, 0.0)
    s = jnp.where(validc, s, 0.0)
    total = jnp.sum(h, keepdims=True)
    denom = jnp.maximum(
        jnp.power(h, _RATIO) * jnp.power(total, 1.0 - _RATIO), 1.0
    )
    out_ref[...] = -jnp.sum(s / denom, keepdims=True) / _C


def kernel(pred, prob):
    del pred
    mesh = plsc.VectorSubcoreMesh(core_axis_name="c", subcore_axis_name="s")
    sc = pl.kernel(
        _sc_body,
        out_type=[
            jax.ShapeDtypeStruct((_NW, 32), jnp.float32),
            jax.ShapeDtypeStruct((_NW, 32), jnp.float32),
        ],
        mesh=mesh,
        compiler_params=pltpu.CompilerParams(needs_layout_passes=False),
        scratch_types=[
            pltpu.VMEM((_C, _RCH, _W), jnp.float32),
            pltpu.VMEM((_C, _RCH, _W), jnp.float32),
            pltpu.VMEM((512,), jnp.float32),
            pltpu.VMEM((512,), jnp.float32),
            pltpu.VMEM((32,), jnp.float32),
            pltpu.VMEM((32,), jnp.float32),
            pltpu.SemaphoreType.DMA,
            pltpu.SemaphoreType.DMA,
        ],
    )
    cnt, s2 = sc(prob)
    nblk = (_H - _RSC) // _HB
    tc_part = pl.pallas_call(
        _tc_hist_body,
        grid=(nblk,),
        in_specs=[
            pl.BlockSpec((1, _C, _HB, _W), lambda i: (0, 0, _RSC // _HB + i, 0))
        ],
        out_specs=pl.BlockSpec((2, 32), lambda i: (0, 0)),
        out_shape=jax.ShapeDtypeStruct((2, 32), jnp.float32),
    )(prob)
    loss = pl.pallas_call(
        _fin_body,
        out_shape=jax.ShapeDtypeStruct((1, 1), jnp.float32),
    )(cnt, s2, tc_part)
    return loss.reshape(())


# speedup vs baseline: 3.1209x; 1.0490x over previous
import functools

import jax
import jax.numpy as jnp
from jax import lax
from jax.experimental import pallas as pl
from jax.experimental.pallas import tpu as pltpu
from jax.experimental.pallas import tpu_sc as plsc

_C = 19
_H = 512
_W = 512
_NW = 32
_RSC = 192
_RW = _RSC // _NW
_RCH = 2
_NCHUNK = _RW // _RCH
_HB = 64
_RATIO = 0.2
_IGNORE = -1.0


def _sc_body(prob_hbm, cnt_out, s2_out, buf0, buf1, acc_cnt, acc_s2,
             obuf_cnt, obuf_s2, sem0, sem1):
    wid = lax.axis_index("s") * 2 + lax.axis_index("c")
    zero16 = jnp.zeros((16,), jnp.float32)
    for j in range(32):
        acc_cnt[pl.ds(j * 16, 16)] = zero16
        acc_s2[pl.ds(j * 16, 16)] = zero16
    lane = lax.iota(jnp.int32, 16)
    ones16 = jnp.ones((16,), jnp.float32)
    base_row = wid * _RW

    bufs = (buf0, buf1)
    sems = (sem0, sem1)

    def copy_of(ci):
        return pltpu.make_async_copy(
            prob_hbm.at[0, :, pl.ds(base_row + ci * _RCH, _RCH), :],
            bufs[ci % 2],
            sems[ci % 2],
        )

    def process16(buf, r, off):
        vals = [buf[c, r, pl.ds(off, 16)] for c in range(_C)]
        sq = [v * v for v in vals]
        while len(sq) > 1:
            nxt = [sq[i] + sq[i + 1] for i in range(0, len(sq) - 1, 2)]
            if len(sq) % 2:
                nxt.append(sq[-1])
            sq = nxt
        s2 = sq[0]
        ent = [(vals[c], c) for c in range(_C)]
        while len(ent) > 1:
            nxt = []
            for i in range(0, len(ent) - 1, 2):
                ml, al = ent[i]
                mr, ar = ent[i + 1]
                gt = mr > ml
                nxt.append((jnp.where(gt, mr, ml), jnp.where(gt, ar, al)))
            if len(ent) % 2:
                nxt.append(ent[-1])
            ent = nxt
        m, a = ent[0]
        valid = m != _IGNORE
        addr = a * 16 + lane
        plsc.addupdate_scatter(acc_cnt, [addr], ones16, mask=valid)
        plsc.addupdate_scatter(acc_s2, [addr], s2, mask=valid)

    copy_of(0).start()
    for ci in range(_NCHUNK):
        if ci + 1 < _NCHUNK:
            copy_of(ci + 1).start()
        copy_of(ci).wait()
        buf = bufs[ci % 2]
        for r in range(_RCH):
            def group_body(g, carry):
                off = g * 64
                process16(buf, r, off)
                process16(buf, r, off + 16)
                process16(buf, r, off + 32)
                process16(buf, r, off + 48)
                return carry

            lax.fori_loop(0, _W // 64, group_body, 0)

    for half in range(2):
        kidx = (lane + half * 16) * 16
        csum = jnp.zeros((16,), jnp.float32)
        ssum = jnp.zeros((16,), jnp.float32)
        for j in range(16):
            csum = csum + plsc.load_gather(acc_cnt, [kidx + j])
            ssum = ssum + plsc.load_gather(acc_s2, [kidx + j])
        obuf_cnt[pl.ds(half * 16, 16)] = csum
        obuf_s2[pl.ds(half * 16, 16)] = ssum
    pltpu.sync_copy(obuf_cnt, cnt_out.at[wid])
    pltpu.sync_copy(obuf_s2, s2_out.at[wid])


def _tc_hist_body(x_ref, out_ref):
    v = x_ref[0, 0]
    m = v
    a = jnp.zeros((_HB, _W), jnp.int32)
    s2 = v * v
    for c in range(1, _C):
        v = x_ref[0, c]
        gt = v > m
        m = jnp.where(gt, v, m)
        a = jnp.where(gt, c, a)
        s2 = s2 + v * v
    valid = m != _IGNORE
    col = lax.broadcasted_iota(jnp.int32, (1, 32), 1)
    acc_c = jnp.zeros((1, 32), jnp.float32)
    acc_s = jnp.zeros((1, 32), jnp.float32)
    for k in range(_C):
        mk = jnp.logical_and(a == k, valid)
        ck = jnp.sum(jnp.where(mk, 1.0, 0.0))
        sk = jnp.sum(jnp.where(mk, s2, 0.0))
        acc_c = acc_c + jnp.where(col == k, ck, 0.0)
        acc_s = acc_s + jnp.where(col == k, sk, 0.0)

    @pl.when(pl.program_id(0) == 0)
    def _():
        out_ref[...] = jnp.zeros_like(out_ref)

    out_ref[0:1, :] = out_ref[0:1, :] + acc_c
    out_ref[1:2, :] = out_ref[1:2, :] + acc_s


def _fin_body(cnt_ref, s2_ref, tc_ref, out_ref):
    h = jnp.sum(cnt_ref[...], axis=0, keepdims=True) + tc_ref[0:1, :]
    s = jnp.sum(s2_ref[...], axis=0, keepdims=True) + tc_ref[1:2, :]
    col = lax.broadcasted_iota(jnp.int32, (1, 32), 1)
    validc = col < _C
    h = jnp.where(validc, h, 0.0)
    s = jnp.where(validc, s, 0.0)
    total = jnp.sum(h, keepdims=True)
    denom = jnp.maximum(
        jnp.power(h, _RATIO) * jnp.power(total, 1.0 - _RATIO), 1.0
    )
    out_ref[...] = -jnp.sum(s / denom, keepdims=True) / _C


def kernel(pred, prob):
    del pred
    mesh = plsc.VectorSubcoreMesh(core_axis_name="c", subcore_axis_name="s")
    sc = pl.kernel(
        _sc_body,
        out_type=[
            jax.ShapeDtypeStruct((_NW, 32), jnp.float32),
            jax.ShapeDtypeStruct((_NW, 32), jnp.float32),
        ],
        mesh=mesh,
        compiler_params=pltpu.CompilerParams(needs_layout_passes=False),
        scratch_types=[
            pltpu.VMEM((_C, _RCH, _W), jnp.float32),
            pltpu.VMEM((_C, _RCH, _W), jnp.float32),
            pltpu.VMEM((512,), jnp.float32),
            pltpu.VMEM((512,), jnp.float32),
            pltpu.VMEM((32,), jnp.float32),
            pltpu.VMEM((32,), jnp.float32),
            pltpu.SemaphoreType.DMA,
            pltpu.SemaphoreType.DMA,
        ],
    )
    cnt, s2 = sc(prob)
    nblk = (_H - _RSC) // _HB
    tc_part = pl.pallas_call(
        _tc_hist_body,
        grid=(nblk,),
        in_specs=[
            pl.BlockSpec((1, _C, _HB, _W), lambda i: (0, 0, _RSC // _HB + i, 0))
        ],
        out_specs=pl.BlockSpec((2, 32), lambda i: (0, 0)),
        out_shape=jax.ShapeDtypeStruct((2, 32), jnp.float32),
    )(prob)
    loss = pl.pallas_call(
        _fin_body,
        out_shape=jax.ShapeDtypeStruct((1, 1), jnp.float32),
    )(cnt, s2, tc_part)
    return loss.reshape(())


# speedup vs baseline: 3.3123x; 1.0613x over previous
import functools

import jax
import jax.numpy as jnp
from jax import lax
from jax.experimental import pallas as pl
from jax.experimental.pallas import tpu as pltpu
from jax.experimental.pallas import tpu_sc as plsc

_C = 19
_H = 512
_W = 512
_NW = 32
_RSC = 128
_RW = _RSC // _NW
_RCH = 2
_NCHUNK = _RW // _RCH
_HB = 128
_NBLK = (_H - _RSC) // _HB
_RATIO = 0.2
_IGNORE = -1.0


def _sc_body(prob_hbm, cnt_out, s2_out, buf0, buf1, acc_cnt, acc_s2,
             obuf_cnt, obuf_s2, sem0, sem1):
    wid = lax.axis_index("s") * 2 + lax.axis_index("c")
    zero16 = jnp.zeros((16,), jnp.float32)
    for j in range(32):
        acc_cnt[pl.ds(j * 16, 16)] = zero16
        acc_s2[pl.ds(j * 16, 16)] = zero16
    lane = lax.iota(jnp.int32, 16)
    ones16 = jnp.ones((16,), jnp.float32)
    base_row = wid * _RW

    bufs = (buf0, buf1)
    sems = (sem0, sem1)

    def copy_of(ci):
        return pltpu.make_async_copy(
            prob_hbm.at[0, :, pl.ds(base_row + ci * _RCH, _RCH), :],
            bufs[ci % 2],
            sems[ci % 2],
        )

    def process16(buf, r, off):
        vals = [buf[c, r, pl.ds(off, 16)] for c in range(_C)]
        sq = [v * v for v in vals]
        while len(sq) > 1:
            nxt = [sq[i] + sq[i + 1] for i in range(0, len(sq) - 1, 2)]
            if len(sq) % 2:
                nxt.append(sq[-1])
            sq = nxt
        s2 = sq[0]
        ent = [(vals[c], c) for c in range(_C)]
        while len(ent) > 1:
            nxt = []
            for i in range(0, len(ent) - 1, 2):
                ml, al = ent[i]
                mr, ar = ent[i + 1]
                gt = mr > ml
                nxt.append((jnp.where(gt, mr, ml), jnp.where(gt, ar, al)))
            if len(ent) % 2:
                nxt.append(ent[-1])
            ent = nxt
        m, a = ent[0]
        valid = m != _IGNORE
        addr = a * 16 + lane
        plsc.addupdate_scatter(acc_cnt, [addr], ones16, mask=valid)
        plsc.addupdate_scatter(acc_s2, [addr], s2, mask=valid)

    copy_of(0).start()
    for ci in range(_NCHUNK):
        if ci + 1 < _NCHUNK:
            copy_of(ci + 1).start()
        copy_of(ci).wait()
        buf = bufs[ci % 2]
        for r in range(_RCH):
            def group_body(g, carry):
                off = g * 64
                process16(buf, r, off)
                process16(buf, r, off + 16)
                process16(buf, r, off + 32)
                process16(buf, r, off + 48)
                return carry

            lax.fori_loop(0, _W // 64, group_body, 0)

    for half in range(2):
        kidx = (lane + half * 16) * 16
        csum = jnp.zeros((16,), jnp.float32)
        ssum = jnp.zeros((16,), jnp.float32)
        for j in range(16):
            csum = csum + plsc.load_gather(acc_cnt, [kidx + j])
            ssum = ssum + plsc.load_gather(acc_s2, [kidx + j])
        obuf_cnt[pl.ds(half * 16, 16)] = csum
        obuf_s2[pl.ds(half * 16, 16)] = ssum
    pltpu.sync_copy(obuf_cnt, cnt_out.at[wid])
    pltpu.sync_copy(obuf_s2, s2_out.at[wid])


def _tc_hist_body(x_ref, out_ref, acc_ref):
    v = x_ref[0, 0]
    m = v
    a = jnp.zeros((_HB, _W), jnp.int32)
    s2 = v * v
    for c in range(1, _C):
        v = x_ref[0, c]
        gt = v > m
        m = jnp.where(gt, v, m)
        a = jnp.where(gt, c, a)
        s2 = s2 + v * v
    a = jnp.where(m != _IGNORE, a, -1)

    @pl.when(pl.program_id(0) == 0)
    def _():
        acc_ref[...] = jnp.zeros_like(acc_ref)

    for k in range(_C):
        mk = a == k
        crow = jnp.sum(jnp.where(mk, 1.0, 0.0), axis=0, keepdims=True)
        srow = jnp.sum(jnp.where(mk, s2, 0.0), axis=0, keepdims=True)
        acc_ref[k:k + 1, :] = acc_ref[k:k + 1, :] + crow
        acc_ref[_C + k:_C + k + 1, :] = acc_ref[_C + k:_C + k + 1, :] + srow

    @pl.when(pl.program_id(0) == _NBLK - 1)
    def _():
        col = lax.broadcasted_iota(jnp.int32, (1, 32), 1)
        acc_c = jnp.zeros((1, 32), jnp.float32)
        acc_s = jnp.zeros((1, 32), jnp.float32)
        for k in range(_C):
            ck = jnp.sum(acc_ref[k:k + 1, :])
            sk = jnp.sum(acc_ref[_C + k:_C + k + 1, :])
            acc_c = acc_c + jnp.where(col == k, ck, 0.0)
            acc_s = acc_s + jnp.where(col == k, sk, 0.0)
        out_ref[0:1, :] = acc_c
        out_ref[1:2, :] = acc_s


def _fin_body(cnt_ref, s2_ref, tc_ref, out_ref):
    h = jnp.sum(cnt_ref[...], axis=0, keepdims=True) + tc_ref[0:1, :]
    s = jnp.sum(s2_ref[...], axis=0, keepdims=True) + tc_ref[1:2, :]
    col = lax.broadcasted_iota(jnp.int32, (1, 32), 1)
    validc = col < _C
    h = jnp.where(validc, h, 0.0)
    s = jnp.where(validc, s, 0.0)
    total = jnp.sum(h, keepdims=True)
    denom = jnp.maximum(
        jnp.power(h, _RATIO) * jnp.power(total, 1.0 - _RATIO), 1.0
    )
    out_ref[...] = -jnp.sum(s / denom, keepdims=True) / _C


def kernel(pred, prob):
    del pred
    mesh = plsc.VectorSubcoreMesh(core_axis_name="c", subcore_axis_name="s")
    sc = pl.kernel(
        _sc_body,
        out_type=[
            jax.ShapeDtypeStruct((_NW, 32), jnp.float32),
            jax.ShapeDtypeStruct((_NW, 32), jnp.float32),
        ],
        mesh=mesh,
        compiler_params=pltpu.CompilerParams(needs_layout_passes=False),
        scratch_types=[
            pltpu.VMEM((_C, _RCH, _W), jnp.float32),
            pltpu.VMEM((_C, _RCH, _W), jnp.float32),
            pltpu.VMEM((512,), jnp.float32),
            pltpu.VMEM((512,), jnp.float32),
            pltpu.VMEM((32,), jnp.float32),
            pltpu.VMEM((32,), jnp.float32),
            pltpu.SemaphoreType.DMA,
            pltpu.SemaphoreType.DMA,
        ],
    )
    cnt, s2 = sc(prob)
    tc_part = pl.pallas_call(
        _tc_hist_body,
        grid=(_NBLK,),
        in_specs=[
            pl.BlockSpec((1, _C, _HB, _W), lambda i: (0, 0, _RSC // _HB + i, 0))
        ],
        out_specs=pl.BlockSpec((2, 32), lambda i: (0, 0)),
        out_shape=jax.ShapeDtypeStruct((2, 32), jnp.float32),
        scratch_shapes=[pltpu.VMEM((2 * _C + 2, _W), jnp.float32)],
    )(prob)
    loss = pl.pallas_call(
        _fin_body,
        out_shape=jax.ShapeDtypeStruct((1, 1), jnp.float32),
    )(cnt, s2, tc_part)
    return loss.reshape(())


# speedup vs baseline: 3.3126x; 1.0001x over previous
import functools

import jax
import jax.numpy as jnp
from jax import lax
from jax.experimental import pallas as pl
from jax.experimental.pallas import tpu as pltpu
from jax.experimental.pallas import tpu_sc as plsc

_C = 19
_H = 512
_W = 512
_NW = 32
_RSC = 128
_RW = _RSC // _NW
_RCH = 2
_NCHUNK = _RW // _RCH
_HB = 128
_NBLK = (_H - _RSC) // _HB
_RATIO = 0.2
_IGNORE = -1.0


def _sc_body(prob_hbm, cnt_out, s2_out, buf0, buf1, acc_cnt, acc_s2,
             obuf_cnt, obuf_s2, sem0, sem1):
    wid = lax.axis_index("s") * 2 + lax.axis_index("c")
    zero16 = jnp.zeros((16,), jnp.float32)
    for j in range(32):
        acc_cnt[pl.ds(j * 16, 16)] = zero16
        acc_s2[pl.ds(j * 16, 16)] = zero16
    lane = lax.iota(jnp.int32, 16)
    ones16 = jnp.ones((16,), jnp.float32)
    base_row = wid * _RW

    bufs = (buf0, buf1)
    sems = (sem0, sem1)

    def copy_of(ci):
        return pltpu.make_async_copy(
            prob_hbm.at[0, :, pl.ds(base_row + ci * _RCH, _RCH), :],
            bufs[ci % 2],
            sems[ci % 2],
        )

    def process16(buf, r, off):
        vals = [buf[c, r, pl.ds(off, 16)] for c in range(_C)]
        sq = [v * v for v in vals]
        while len(sq) > 1:
            nxt = [sq[i] + sq[i + 1] for i in range(0, len(sq) - 1, 2)]
            if len(sq) % 2:
                nxt.append(sq[-1])
            sq = nxt
        s2 = sq[0]
        ent = [(vals[c], c) for c in range(_C)]
        while len(ent) > 1:
            nxt = []
            for i in range(0, len(ent) - 1, 2):
                ml, al = ent[i]
                mr, ar = ent[i + 1]
                gt = mr > ml
                nxt.append((jnp.where(gt, mr, ml), jnp.where(gt, ar, al)))
            if len(ent) % 2:
                nxt.append(ent[-1])
            ent = nxt
        m, a = ent[0]
        valid = m != _IGNORE
        addr = a * 16 + lane
        plsc.addupdate_scatter(acc_cnt, [addr], ones16, mask=valid)
        plsc.addupdate_scatter(acc_s2, [addr], s2, mask=valid)

    copy_of(0).start()
    for ci in range(_NCHUNK):
        if ci + 1 < _NCHUNK:
            copy_of(ci + 1).start()
        copy_of(ci).wait()
        buf = bufs[ci % 2]
        for r in range(_RCH):
            def group_body(g, carry):
                off = g * 64
                process16(buf, r, off)
                process16(buf, r, off + 16)
                process16(buf, r, off + 32)
                process16(buf, r, off + 48)
                return carry

            lax.fori_loop(0, _W // 64, group_body, 0)

    for half in range(2):
        kidx = (lane + half * 16) * 16
        csum = jnp.zeros((16,), jnp.float32)
        ssum = jnp.zeros((16,), jnp.float32)
        for j in range(16):
            csum = csum + plsc.load_gather(acc_cnt, [kidx + j])
            ssum = ssum + plsc.load_gather(acc_s2, [kidx + j])
        obuf_cnt[pl.ds(half * 16, 16)] = csum
        obuf_s2[pl.ds(half * 16, 16)] = ssum
    pltpu.sync_copy(obuf_cnt, cnt_out.at[wid])
    pltpu.sync_copy(obuf_s2, s2_out.at[wid])


_SUB = 16


def _tc_hist_body(x_ref, out_ref, acc_ref):
    def body(sub, carry):
        acc_c, acc_s = carry
        r0 = sub * _SUB
        v = x_ref[0, 0, pl.ds(r0, _SUB), :]
        m = v
        a = jnp.zeros((_SUB, _W), jnp.int32)
        s2 = v * v
        for c in range(1, _C):
            v = x_ref[0, c, pl.ds(r0, _SUB), :]
            gt = v > m
            m = jnp.where(gt, v, m)
            a = jnp.where(gt, c, a)
            s2 = s2 + v * v
        a = jnp.where(m != _IGNORE, a, -1)
        crows = []
        srows = []
        for k in range(_C):
            mk = a == k
            crows.append(jnp.sum(jnp.where(mk, 1.0, 0.0), axis=0, keepdims=True))
            srows.append(jnp.sum(jnp.where(mk, s2, 0.0), axis=0, keepdims=True))
        return (acc_c + jnp.concatenate(crows, axis=0),
                acc_s + jnp.concatenate(srows, axis=0))

    z = jnp.zeros((_C, _W), jnp.float32)
    acc_c, acc_s = lax.fori_loop(0, _HB // _SUB, body, (z, z))

    @pl.when(pl.program_id(0) == 0)
    def _():
        acc_ref[...] = jnp.zeros_like(acc_ref)

    acc_ref[0:_C, :] = acc_ref[0:_C, :] + acc_c
    acc_ref[24:24 + _C, :] = acc_ref[24:24 + _C, :] + acc_s

    @pl.when(pl.program_id(0) == _NBLK - 1)
    def _():
        col = lax.broadcasted_iota(jnp.int32, (1, 32), 1)
        out_c = jnp.zeros((1, 32), jnp.float32)
        out_s = jnp.zeros((1, 32), jnp.float32)
        for k in range(_C):
            ck = jnp.sum(acc_ref[k:k + 1, :])
            sk = jnp.sum(acc_ref[24 + k:24 + k + 1, :])
            out_c = out_c + jnp.where(col == k, ck, 0.0)
            out_s = out_s + jnp.where(col == k, sk, 0.0)
        out_ref[0:1, :] = out_c
        out_ref[1:2, :] = out_s


def _fin_body(cnt_ref, s2_ref, tc_ref, out_ref):
    h = jnp.sum(cnt_ref[...], axis=0, keepdims=True) + tc_ref[0:1, :]
    s = jnp.sum(s2_ref[...], axis=0, keepdims=True) + tc_ref[1:2, :]
    col = lax.broadcasted_iota(jnp.int32, (1, 32), 1)
    validc = col < _C
    h = jnp.where(validc, h, 0.0)
    s = jnp.where(validc, s, 0.0)
    total = jnp.sum(h, keepdims=True)
    denom = jnp.maximum(
        jnp.power(h, _RATIO) * jnp.power(total, 1.0 - _RATIO), 1.0
    )
    out_ref[...] = -jnp.sum(s / denom, keepdims=True) / _C


def kernel(pred, prob):
    del pred
    mesh = plsc.VectorSubcoreMesh(core_axis_name="c", subcore_axis_name="s")
    sc = pl.kernel(
        _sc_body,
        out_type=[
            jax.ShapeDtypeStruct((_NW, 32), jnp.float32),
            jax.ShapeDtypeStruct((_NW, 32), jnp.float32),
        ],
        mesh=mesh,
        compiler_params=pltpu.CompilerParams(needs_layout_passes=False),
        scratch_types=[
            pltpu.VMEM((_C, _RCH, _W), jnp.float32),
            pltpu.VMEM((_C, _RCH, _W), jnp.float32),
            pltpu.VMEM((512,), jnp.float32),
            pltpu.VMEM((512,), jnp.float32),
            pltpu.VMEM((32,), jnp.float32),
            pltpu.VMEM((32,), jnp.float32),
            pltpu.SemaphoreType.DMA,
            pltpu.SemaphoreType.DMA,
        ],
    )
    cnt, s2 = sc(prob)
    tc_part = pl.pallas_call(
        _tc_hist_body,
        grid=(_NBLK,),
        in_specs=[
            pl.BlockSpec((1, _C, _HB, _W), lambda i: (0, 0, _RSC // _HB + i, 0))
        ],
        out_specs=pl.BlockSpec((2, 32), lambda i: (0, 0)),
        out_shape=jax.ShapeDtypeStruct((2, 32), jnp.float32),
        scratch_shapes=[pltpu.VMEM((48, _W), jnp.float32)],
    )(prob)
    loss = pl.pallas_call(
        _fin_body,
        out_shape=jax.ShapeDtypeStruct((1, 1), jnp.float32),
    )(cnt, s2, tc_part)
    return loss.reshape(())
